# baseline (speedup 1.0000x reference)
import jax
import jax.numpy as jnp
from jax import lax
from jax.experimental import pallas as pl
from jax.experimental.pallas import tpu as pltpu
from jax.experimental.pallas import tpu_sc as plsc

B = 16384
S = 200
V = 1000000
D = 64
C = 16

NC, NS = 2, 16
NW = NC * NS
RPW = B // NW
TS = 50
NSLAB = S // TS

MBLK = 16384
NBLK = (V + MBLK - 1) // MBLK
VP = NBLK * MBLK


def _mm_body(t_ref, w_ref, o_ref):
    res = lax.dot_general(t_ref[...].astype(jnp.bfloat16),
                          w_ref[...].astype(jnp.bfloat16),
                          (((0,), (0,)), ((), ())),
                          preferred_element_type=jnp.float32)
    for j in range(8):
        o_ref[:, j * C:(j + 1) * C] = res[j * (MBLK // 8):(j + 1) * (MBLK // 8), :]


def _project_table(table_t, w_t):
    return pl.pallas_call(
        _mm_body,
        grid=(NBLK,),
        in_specs=[
            pl.BlockSpec((D, MBLK), lambda i: (0, i)),
            pl.BlockSpec((D, C), lambda i: (0, 0)),
        ],
        out_specs=pl.BlockSpec((MBLK // 8, 8 * C), lambda i: (i, 0)),
        out_shape=jax.ShapeDtypeStruct((VP // 8, 8 * C), jnp.float32),
    )(table_t, w_t)


def _sc_body(y_hbm, len_hbm, p_hbm, b_hbm, out_hbm,
             xv0, xv1, rb0, rb1, accv, outv, lenv, biasv,
             sr0, sr1, sx0, sx1):
    xvs = (xv0, xv1)
    rbufs = (rb0, rb1)
    sems = (sr0, sr1)
    xsems = (sx0, sx1)
    wid = lax.axis_index("c") * NS + lax.axis_index("s")
    base = wid * RPW

    pltpu.sync_copy(b_hbm, biasv)
    pltpu.sync_copy(len_hbm.at[pl.ds(base, RPW)], lenv)
    bias = biasv[...]

    def zbody(i, _):
        z = jnp.zeros((16,), jnp.float32)
        for u in range(4):
            accv[i * 4 + u, :] = z
        return 0

    lax.fori_loop(0, RPW // 4, zbody, 0)

    NP = TS // 2

    def stage(sc, xq):
        for h in range(2):
            pltpu.async_copy(
                y_hbm.at[pl.ds(sc * NP, NP), pl.ds(h * B + base, RPW)],
                xvs[xq].at[:, pl.ds(h * RPW, RPW)],
                xsems[xq])

    def wait_x(xq):
        for h in range(2):
            pltpu.make_async_copy(
                y_hbm.at[pl.ds(0, NP), pl.ds(h * B + base, RPW)],
                xvs[xq].at[:, pl.ds(h * RPW, RPW)], xsems[xq]).wait()

    def issue(xq, prow, q):
        pltpu.async_copy(p_hbm.at[xvs[xq].at[prow]], rbufs[q], sems[q])

    def drain(q):
        pltpu.make_async_copy(p_hbm.at[xv0.at[0]], rbufs[q], sems[q]).wait()

    def accumulate(q):
        buf = rbufs[q]

        def abody(i, _):
            for u in range(16):
                r = i * 16 + u
                plsc.addupdate(accv.at[r], buf[r, :] + buf[RPW + r, :])
            return 0

        lax.fori_loop(0, RPW // 16, abody, 0)

    stage(0, 0)
    wait_x(0)
    issue(0, 0, 0)
    issue(0, 1, 1)
    stage(1, 1)

    def slab_pair(p_, _):
        for half in range(2):
            sc = p_ * 2 + half
            xq, nxq = half, 1 - half
            par = half

            def mbody(kk, _):
                for i01 in range(2):
                    p = kk * 2 + i01
                    q = (i01 + par) % 2
                    drain(q)
                    accumulate(q)
                    issue(xq, p + 2, q)
                return 0

            lax.fori_loop(0, (NP - 3) // 2, mbody, 0)

            q22 = (22 + par) % 2
            drain(q22)
            accumulate(q22)
            issue(xq, 24, q22)

            for pp in (23, 24):
                q = (pp + par) % 2
                drain(q)
                accumulate(q)

                @pl.when(sc + 1 < NSLAB)
                def _(q=q, nxq=nxq, pp=pp):
                    if pp == 23:
                        wait_x(nxq)
                    issue(nxq, pp - 23, q)

            @pl.when(sc + 2 < NSLAB)
            def _(sc=sc, xq=xq):
                stage(sc + 2, xq)
        return 0

    lax.fori_loop(0, NSLAB // 2, slab_pair, 0)

    def ebody(i, _):
        for u in range(4):
            r = i * 4 + u
            outv[r, :] = accv[r, :] / lenv[r, :] + bias
        return 0

    lax.fori_loop(0, RPW // 4, ebody, 0)
    pltpu.sync_copy(outv, out_hbm.at[pl.ds(base, RPW)])


def _pooled_logits(y, len_bcast, p, b):
    mesh = plsc.VectorSubcoreMesh(core_axis_name="c", subcore_axis_name="s")
    call = pl.kernel(
        _sc_body,
        out_type=jax.ShapeDtypeStruct((B, C), jnp.float32),
        mesh=mesh,
        scratch_types=[
            pltpu.VMEM((TS // 2, 2 * RPW), jnp.int32),
            pltpu.VMEM((TS // 2, 2 * RPW), jnp.int32),
            pltpu.VMEM((2 * RPW, C), jnp.float32),
            pltpu.VMEM((2 * RPW, C), jnp.float32),
            pltpu.VMEM((RPW, C), jnp.float32),
            pltpu.VMEM((RPW, C), jnp.float32),
            pltpu.VMEM((RPW, C), jnp.float32),
            pltpu.VMEM((C,), jnp.float32),
            pltpu.SemaphoreType.DMA,
            pltpu.SemaphoreType.DMA,
            pltpu.SemaphoreType.DMA,
            pltpu.SemaphoreType.DMA,
        ],
        compiler_params=pltpu.CompilerParams(use_tc_tiling_on_sc=False),
    )
    return call(y, len_bcast, p, b)


def kernel(x, lengths, table, W, b):
    p = _project_table(table.T, W.T).reshape(VP, C)
    lg = MBLK.bit_length() - 1
    q = MBLK // 8
    x_lin = ((x >> lg) << lg) | ((x & (q - 1)) << 3) | ((x >> (lg - 3)) & 7)
    y = x_lin.T.reshape(S // 2, 2 * B)
    lb2d = lax.optimization_barrier(jnp.broadcast_to(
        lengths.astype(jnp.float32).reshape(B // 8, 8, 1),
        (B // 8, 8, C)).reshape(B // 8, 8 * C))
    len_bcast = lb2d.reshape(B, C)
    return _pooled_logits(y, len_bcast, p, b)
